# baseline (speedup 1.0000x reference)
import functools

import jax
import jax.numpy as jnp
import numpy as np
from jax import lax
from jax.experimental import pallas as pl
from jax.experimental.pallas import tpu as pltpu
from jax.experimental.pallas import tpu_sc as plsc

F = 26
D = 16
B = 4096
TOTAL = 26000
NWORD = F // 2
ROW = 256

_SC = plsc.get_sparse_core_info()
NC, NS = _SC.num_cores, _SC.num_subcores
NW = NC * NS
SPT = B // NW
CH = 4
NCHUNK = SPT // CH
IDXC = CH * F
HI = jnp.int32(-65536)


def _lane_sum(v, lanes):
    for step in (8, 4, 2, 1):
        v = v + v.at[lanes ^ step].get(mode="promise_in_bounds", unique_indices=True)
    return v


def _sc_body(p_hbm, idx_hbm, bias_hbm, out_hbm, idx_v, rows_v,
             part_v, out_v, bias_v, sem0, sem1):
    wid = lax.axis_index("s") * NC + lax.axis_index("c")
    base = wid * SPT
    pltpu.sync_copy(idx_hbm.at[pl.ds(base * F, SPT * F)], idx_v)
    pltpu.sync_copy(bias_hbm, bias_v)

    def start(c, half, sem):
        pltpu.async_copy(
            p_hbm.at[idx_v.at[pl.ds(c * IDXC, IDXC)]],
            rows_v.at[pl.ds(half * IDXC, IDXC)], sem)

    start(0, 0, sem0)
    lanes = lax.iota(jnp.int32, 16)
    bias_b = _lane_sum(bias_v[...], lanes)

    def chunk(c, carry):
        nxt = c + 1

        @pl.when(jnp.logical_and(nxt < NCHUNK, nxt % 2 == 0))
        def _():
            start(nxt, 0, sem0)

        @pl.when(jnp.logical_and(nxt < NCHUNK, nxt % 2 == 1))
        def _():
            start(nxt, 1, sem1)

        @pl.when(c % 2 == 0)
        def _():
            pltpu.make_async_copy(
                p_hbm, rows_v.at[pl.ds(0, IDXC)], sem0).wait()

        @pl.when(c % 2 == 1)
        def _():
            pltpu.make_async_copy(
                p_hbm, rows_v.at[pl.ds(0, IDXC)], sem1).wait()

        roff = (c % 2) * IDXC

        def sample(s, carry2):
            r0 = roff + s * F

            def word(r, k):
                w = rows_v[r, pl.ds(k * D, D)]
                return (lax.bitcast_convert_type(w << 16, jnp.float32),
                        lax.bitcast_convert_type(w & HI, jnp.float32))

            acc = [jnp.zeros((D,), jnp.float32) for _ in range(2)]
            for g in range(NWORD):
                u0, u1 = word(r0 + 2 * g, g)
                v0, v1 = word(r0 + 2 * g + 1, g)
                acc[0] = acc[0] + u0 + v1
                acc[1] = acc[1] + u1 * v0
            for ga in range(NWORD):
                for gb in range(ga + 1, NWORD):
                    x0, x1 = word(r0 + 2 * ga, gb)
                    y0, y1 = word(r0 + 2 * ga + 1, gb)
                    z0, z1 = word(r0 + 2 * gb, ga)
                    w0, w1 = word(r0 + 2 * gb + 1, ga)
                    acc[0] = acc[0] + x0 * z0
                    acc[1] = acc[1] + x1 * w0
                    acc[0] = acc[0] + y0 * z1
                    acc[1] = acc[1] + y1 * w1
            part_v[(c % 4) * CH + s] = acc[0] + acc[1]
            return carry2

        lax.fori_loop(0, CH, sample, carry)

        @pl.when(c % 4 == 3)
        def _():
            vec = jnp.zeros((D,), jnp.float32)
            for l in range(D):
                sm = _lane_sum(part_v[l], lanes)
                vec = jnp.where(lanes == l, sm, vec)
            vec = vec + bias_b
            out_v[pl.ds((c // 4) * D, D)] = 1.0 / (1.0 + jnp.exp(-vec))

        return carry

    lax.fori_loop(0, NCHUNK, chunk, 0)
    pltpu.sync_copy(out_v, out_hbm.at[pl.ds(base, SPT)])


@functools.partial(
    pl.kernel,
    mesh=plsc.VectorSubcoreMesh(core_axis_name="c", subcore_axis_name="s"),
    out_type=jax.ShapeDtypeStruct((B,), jnp.float32),
    scratch_types=[
        pltpu.VMEM((SPT * F,), jnp.int32),
        pltpu.VMEM((2 * IDXC, ROW), jnp.int32),
        pltpu.VMEM((4 * CH, D), jnp.float32),
        pltpu.VMEM((SPT,), jnp.float32),
        pltpu.VMEM((D,), jnp.float32),
        pltpu.SemaphoreType.DMA,
        pltpu.SemaphoreType.DMA,
    ],
)
def _sc_kernel(p_hbm, idx_hbm, bias_hbm, out_hbm, idx_v, rows_v,
               part_v, out_v, bias_v, sem0, sem1):
    _sc_body(p_hbm, idx_hbm, bias_hbm, out_hbm, idx_v, rows_v,
             part_v, out_v, bias_v, sem0, sem1)


def kernel(x, W_linear, W_cross, bias):
    offs = jnp.arange(F, dtype=jnp.int32) * 1000
    idx = (x.astype(jnp.int32) + offs[None, :]).reshape(-1)
    p = jnp.transpose(W_cross, (1, 0, 2))
    fi = np.arange(TOTAL) // 1000
    linblock = jnp.concatenate(
        [W_linear.astype(jnp.float32), jnp.zeros((TOTAL, D - 1), jnp.float32)],
        axis=1)
    p = p.at[np.arange(TOTAL), fi].set(linblock)
    pb = p.astype(jnp.bfloat16)
    inter = jnp.stack([pb[:, 0::2, :], pb[:, 1::2, :]], axis=-1)
    w32 = lax.bitcast_convert_type(inter, jnp.int32).reshape(TOTAL, NWORD * D)
    w32 = jnp.concatenate(
        [w32, jnp.zeros((TOTAL, ROW - NWORD * D), jnp.int32)], axis=1)
    bias_pad = jnp.concatenate(
        [bias.astype(jnp.float32), jnp.zeros((D - 1,), jnp.float32)])
    out = _sc_kernel(w32, idx, bias_pad)
    return out.reshape(B, 1)


# speedup vs baseline: 1.0005x; 1.0005x over previous
import functools

import jax
import jax.numpy as jnp
import numpy as np
from jax import lax
from jax.experimental import pallas as pl
from jax.experimental.pallas import tpu as pltpu
from jax.experimental.pallas import tpu_sc as plsc

F = 26
D = 16
B = 4096
TOTAL = 26000
NWORD = F // 2
ROW = 256

_SC = plsc.get_sparse_core_info()
NC, NS = _SC.num_cores, _SC.num_subcores
NW = NC * NS
SPT = B // NW
CH = 4
NCHUNK = SPT // CH
IDXC = CH * F
HI = jnp.int32(-65536)


def _lane_sum(v, lanes):
    for step in (8, 4, 2, 1):
        v = v + v.at[lanes ^ step].get(mode="promise_in_bounds", unique_indices=True)
    return v


def _sc_body(p_hbm, idx_hbm, bias_hbm, out_hbm, idx_v, rows_v,
             part_v, out_v, bias_v, sem0, sem1):
    wid = lax.axis_index("s") * NC + lax.axis_index("c")
    base = wid * SPT
    pltpu.sync_copy(idx_hbm.at[pl.ds(base * F, SPT * F)], idx_v)
    pltpu.sync_copy(bias_hbm, bias_v)

    def start(c, half, sem):
        pltpu.async_copy(
            p_hbm.at[idx_v.at[pl.ds(c * IDXC, IDXC)]],
            rows_v.at[pl.ds(half * IDXC, IDXC)], sem)

    start(0, 0, sem0)
    lanes = lax.iota(jnp.int32, 16)
    bias_b = _lane_sum(bias_v[...], lanes)

    def chunk(c, carry):
        nxt = c + 1

        @pl.when(jnp.logical_and(nxt < NCHUNK, nxt % 2 == 0))
        def _():
            start(nxt, 0, sem0)

        @pl.when(jnp.logical_and(nxt < NCHUNK, nxt % 2 == 1))
        def _():
            start(nxt, 1, sem1)

        @pl.when(c % 2 == 0)
        def _():
            pltpu.make_async_copy(
                p_hbm, rows_v.at[pl.ds(0, IDXC)], sem0).wait()

        @pl.when(c % 2 == 1)
        def _():
            pltpu.make_async_copy(
                p_hbm, rows_v.at[pl.ds(0, IDXC)], sem1).wait()

        roff = (c % 2) * IDXC

        hi_v = jnp.full((D,), -65536, jnp.int32)
        sh_v = jnp.full((D,), 65536, jnp.int32)

        def sample(s, carry2):
            r0 = roff + s * F

            def word(r, k):
                w = rows_v[r, pl.ds(k * D, D)]
                return (lax.bitcast_convert_type(w * sh_v, jnp.float32),
                        lax.bitcast_convert_type(w & hi_v, jnp.float32))

            acc = [jnp.zeros((D,), jnp.float32) for _ in range(2)]
            for g in range(NWORD):
                u0, u1 = word(r0 + 2 * g, g)
                v0, v1 = word(r0 + 2 * g + 1, g)
                acc[0] = acc[0] + u0 + v1
                acc[1] = acc[1] + u1 * v0
            for ga in range(NWORD):
                for gb in range(ga + 1, NWORD):
                    x0, x1 = word(r0 + 2 * ga, gb)
                    y0, y1 = word(r0 + 2 * ga + 1, gb)
                    z0, z1 = word(r0 + 2 * gb, ga)
                    w0, w1 = word(r0 + 2 * gb + 1, ga)
                    acc[0] = acc[0] + x0 * z0
                    acc[1] = acc[1] + x1 * w0
                    acc[0] = acc[0] + y0 * z1
                    acc[1] = acc[1] + y1 * w1
            part_v[(c % 4) * CH + s] = acc[0] + acc[1]
            return carry2

        lax.fori_loop(0, CH, sample, carry)

        @pl.when(c % 4 == 3)
        def _():
            vec = jnp.zeros((D,), jnp.float32)
            for l in range(D):
                sm = _lane_sum(part_v[l], lanes)
                vec = jnp.where(lanes == l, sm, vec)
            vec = vec + bias_b
            out_v[pl.ds((c // 4) * D, D)] = 1.0 / (1.0 + jnp.exp(-vec))

        return carry

    lax.fori_loop(0, NCHUNK, chunk, 0)
    pltpu.sync_copy(out_v, out_hbm.at[pl.ds(base, SPT)])


@functools.partial(
    pl.kernel,
    mesh=plsc.VectorSubcoreMesh(core_axis_name="c", subcore_axis_name="s"),
    out_type=jax.ShapeDtypeStruct((B,), jnp.float32),
    scratch_types=[
        pltpu.VMEM((SPT * F,), jnp.int32),
        pltpu.VMEM((2 * IDXC, ROW), jnp.int32),
        pltpu.VMEM((4 * CH, D), jnp.float32),
        pltpu.VMEM((SPT,), jnp.float32),
        pltpu.VMEM((D,), jnp.float32),
        pltpu.SemaphoreType.DMA,
        pltpu.SemaphoreType.DMA,
    ],
)
def _sc_kernel(p_hbm, idx_hbm, bias_hbm, out_hbm, idx_v, rows_v,
               part_v, out_v, bias_v, sem0, sem1):
    _sc_body(p_hbm, idx_hbm, bias_hbm, out_hbm, idx_v, rows_v,
             part_v, out_v, bias_v, sem0, sem1)


def kernel(x, W_linear, W_cross, bias):
    offs = jnp.arange(F, dtype=jnp.int32) * 1000
    idx = (x.astype(jnp.int32) + offs[None, :]).reshape(-1)
    p = jnp.transpose(W_cross, (1, 0, 2))
    fi = np.arange(TOTAL) // 1000
    linblock = jnp.concatenate(
        [W_linear.astype(jnp.float32), jnp.zeros((TOTAL, D - 1), jnp.float32)],
        axis=1)
    p = p.at[np.arange(TOTAL), fi].set(linblock)
    pb = p.astype(jnp.bfloat16)
    inter = jnp.stack([pb[:, 0::2, :], pb[:, 1::2, :]], axis=-1)
    w32 = lax.bitcast_convert_type(inter, jnp.int32).reshape(TOTAL, NWORD * D)
    w32 = jnp.concatenate(
        [w32, jnp.zeros((TOTAL, ROW - NWORD * D), jnp.int32)], axis=1)
    bias_pad = jnp.concatenate(
        [bias.astype(jnp.float32), jnp.zeros((D - 1,), jnp.float32)])
    out = _sc_kernel(w32, idx, bias_pad)
    return out.reshape(B, 1)


# speedup vs baseline: 24.1139x; 24.1011x over previous
import functools

import jax
import jax.numpy as jnp
import numpy as np
from jax import lax
from jax.experimental import pallas as pl
from jax.experimental.pallas import tpu as pltpu
from jax.experimental.pallas import tpu_sc as plsc

F = 26
D = 16
B = 4096
TOTAL = 26000
NWORD = F // 2
ROW = 256

_SC = plsc.get_sparse_core_info()
NC, NS = _SC.num_cores, _SC.num_subcores
NW = NC * NS
SPT = B // NW
CH = 4
NCHUNK = SPT // CH
IDXC = CH * F
HI = jnp.int32(-65536)


def _lane_sum(v, lanes):
    for step in (8, 4, 2, 1):
        v = v + v.at[lanes ^ step].get(mode="promise_in_bounds", unique_indices=True)
    return v


def _sc_body(p_hbm, idx_hbm, bias_hbm, out_hbm, idx_v, rows_v,
             part_v, out_v, bias_v, sem0, sem1):
    wid = lax.axis_index("s") * NC + lax.axis_index("c")
    base = wid * SPT
    pltpu.sync_copy(idx_hbm.at[pl.ds(base * F, SPT * F)], idx_v)
    pltpu.sync_copy(bias_hbm, bias_v)

    def start(c, half, sem):
        pltpu.async_copy(
            p_hbm.at[idx_v.at[pl.ds(c * IDXC, IDXC)]],
            rows_v.at[pl.ds(half * IDXC, IDXC)], sem)

    start(0, 0, sem0)
    lanes = lax.iota(jnp.int32, 16)
    bias_b = _lane_sum(bias_v[...], lanes)

    def chunk(c, carry):
        nxt = c + 1

        @pl.when(jnp.logical_and(nxt < NCHUNK, nxt % 2 == 0))
        def _():
            start(nxt, 0, sem0)

        @pl.when(jnp.logical_and(nxt < NCHUNK, nxt % 2 == 1))
        def _():
            start(nxt, 1, sem1)

        @pl.when(c % 2 == 0)
        def _():
            pltpu.make_async_copy(
                p_hbm, rows_v.at[pl.ds(0, IDXC)], sem0).wait()

        @pl.when(c % 2 == 1)
        def _():
            pltpu.make_async_copy(
                p_hbm, rows_v.at[pl.ds(0, IDXC)], sem1).wait()

        roff = (c % 2) * IDXC

        hi_v = jnp.full((D,), -65536, jnp.int32)
        sh_v = jnp.full((D,), 65536, jnp.int32)

        def sample(s, carry2):
            r0 = roff + s * F

            def word(r, k):
                w = rows_v[r, pl.ds(k * D, D)]
                return (lax.bitcast_convert_type(w * sh_v, jnp.float32),
                        lax.bitcast_convert_type(w & hi_v, jnp.float32))

            acc = [jnp.zeros((D,), jnp.float32) for _ in range(2)]
            for g in range(NWORD):
                u0, u1 = word(r0 + 2 * g, g)
                v0, v1 = word(r0 + 2 * g + 1, g)
                acc[0] = acc[0] + u0 + v1
                acc[1] = acc[1] + u1 * v0
            for ga in range(NWORD):
                for gb in range(ga + 1, NWORD):
                    x0, x1 = word(r0 + 2 * ga, gb)
                    y0, y1 = word(r0 + 2 * ga + 1, gb)
                    z0, z1 = word(r0 + 2 * gb, ga)
                    w0, w1 = word(r0 + 2 * gb + 1, ga)
                    acc[0] = acc[0] + x0 * z0
                    acc[1] = acc[1] + x1 * w0
                    acc[0] = acc[0] + y0 * z1
                    acc[1] = acc[1] + y1 * w1
            part_v[(c % 4) * CH + s] = acc[0] + acc[1]
            return carry2

        lax.fori_loop(0, CH, sample, carry)

        @pl.when(c % 4 == 3)
        def _():
            vec = jnp.zeros((D,), jnp.float32)
            for l in range(D):
                sm = _lane_sum(part_v[l], lanes)
                vec = jnp.where(lanes == l, sm, vec)
            vec = vec + bias_b
            out_v[pl.ds((c // 4) * D, D)] = 1.0 / (1.0 + jnp.exp(-vec))

        return carry

    lax.fori_loop(0, NCHUNK, chunk, 0)
    pltpu.sync_copy(out_v, out_hbm.at[pl.ds(base, SPT)])


@functools.partial(
    pl.kernel,
    mesh=plsc.VectorSubcoreMesh(core_axis_name="c", subcore_axis_name="s"),
    out_type=jax.ShapeDtypeStruct((B,), jnp.float32),
    scratch_types=[
        pltpu.VMEM((SPT * F,), jnp.int32),
        pltpu.VMEM((2 * IDXC, ROW), jnp.int32),
        pltpu.VMEM((4 * CH, D), jnp.float32),
        pltpu.VMEM((SPT,), jnp.float32),
        pltpu.VMEM((D,), jnp.float32),
        pltpu.SemaphoreType.DMA,
        pltpu.SemaphoreType.DMA,
    ],
)
def _sc_kernel(p_hbm, idx_hbm, bias_hbm, out_hbm, idx_v, rows_v,
               part_v, out_v, bias_v, sem0, sem1):
    _sc_body(p_hbm, idx_hbm, bias_hbm, out_hbm, idx_v, rows_v,
             part_v, out_v, bias_v, sem0, sem1)


def kernel(x, W_linear, W_cross, bias):
    offs = jnp.arange(F, dtype=jnp.int32) * 1000
    idx = (x.astype(jnp.int32) + offs[None, :]).reshape(-1)
    p = jnp.transpose(W_cross, (1, 0, 2))
    fi = np.arange(TOTAL) // 1000
    linblock = jnp.concatenate(
        [W_linear.astype(jnp.float32), jnp.zeros((TOTAL, D - 1), jnp.float32)],
        axis=1)
    own = jnp.asarray(fi[:, None] == np.arange(F)[None, :])
    p = jnp.where(own[:, :, None], linblock[:, None, :], p)
    pb = p.astype(jnp.bfloat16)
    inter = jnp.stack([pb[:, 0::2, :], pb[:, 1::2, :]], axis=-1)
    w32 = lax.bitcast_convert_type(inter, jnp.int32).reshape(TOTAL, NWORD * D)
    w32 = jnp.concatenate(
        [w32, jnp.zeros((TOTAL, ROW - NWORD * D), jnp.int32)], axis=1)
    bias_pad = jnp.concatenate(
        [bias.astype(jnp.float32), jnp.zeros((D - 1,), jnp.float32)])
    out = _sc_kernel(w32, idx, bias_pad)
    return out.reshape(B, 1)
